# Initial kernel scaffold; baseline (speedup 1.0000x reference)
import jax
import jax.numpy as jnp
from jax.experimental import pallas as pl


def kernel(pts, params):
    raise NotImplementedError("write your pallas kernel here")


# speedup vs baseline: 11.8479x; 11.8479x over previous
import jax
import jax.numpy as jnp
from jax.experimental import pallas as pl
from jax.experimental.pallas import tpu as pltpu

_K = 16
_EPS = 1e-5
_F32 = jnp.float32


def _knn_kernel(ptile_ref, ptsb_ref, ptsTb_ref, out_ref, *, tq):
    t = pl.program_id(1)
    ptile = ptile_ref[0]
    ptsb = ptsb_ref[0]
    ptsT = ptsTb_ref[0]
    n = ptsT.shape[1]
    sq_b = jnp.sum(ptsT * ptsT, axis=0, keepdims=True)
    sq_t = jnp.sum(ptile * ptile, axis=1, keepdims=True)
    cross = jax.lax.dot_general(ptile, ptsT, (((1,), (0,)), ((), ())),
                                preferred_element_type=_F32)
    d2 = sq_t + sq_b - 2.0 * cross
    col = jax.lax.broadcasted_iota(jnp.int32, (tq, n), 1)
    row = jax.lax.broadcasted_iota(jnp.int32, (tq, n), 0) + t * tq
    d2 = jnp.where(col == row, jnp.inf, d2)
    slots = []
    for _ in range(_K):
        m = jnp.min(d2, axis=1, keepdims=True)
        idx = jnp.min(jnp.where(d2 == m, col, n), axis=1, keepdims=True)
        sel = col == idx
        d2 = jnp.where(sel, jnp.inf, d2)
        nbr = jax.lax.dot_general(sel.astype(_F32), ptsb,
                                  (((1,), (0,)), ((), ())),
                                  preferred_element_type=_F32)
        dd = jnp.sqrt(jnp.maximum(m, 0.0) + 1e-8)
        slots.append(jnp.concatenate([ptile, nbr - ptile, dd], axis=1))
    out_ref[0] = jnp.concatenate(slots, axis=1)


def _fold(v, groups, width):
    acc = v[:, 0:width]
    for k in range(1, groups):
        acc = acc + v[:, k * width:(k + 1) * width]
    return acc


def _tile_lanes(v, groups):
    return jnp.concatenate([v] * groups, axis=1)


def _nbrmlp_kernel(x_ref, w1_ref, g1_ref, b1_ref, w2_ref, g2_ref, b2_ref,
                   out_ref, s1, q1, s2, q2, *, m_tokens):
    p = pl.program_id(0)
    t = pl.program_id(1)
    x = x_ref[...]
    y1 = jnp.dot(x, w1_ref[...], preferred_element_type=_F32)

    @pl.when(jnp.logical_and(p == 0, t == 0))
    def _():
        s1[...] = jnp.zeros_like(s1)
        q1[...] = jnp.zeros_like(q1)

    @pl.when(p == 0)
    def _():
        s1[...] += jnp.sum(y1, axis=0, keepdims=True)
        q1[...] += jnp.sum(y1 * y1, axis=0, keepdims=True)

    def layer2():
        s1f = _fold(s1[...], _K, 16)
        q1f = _fold(q1[...], _K, 16)
        m1 = s1f / m_tokens
        v1 = q1f / m_tokens - m1 * m1
        m1p = _tile_lanes(m1, _K)
        sc1p = _tile_lanes(1.0 / jnp.sqrt(v1 + _EPS), _K)
        h1 = jnp.maximum((y1 - m1p) * sc1p * g1_ref[...] + b1_ref[...], 0.0)
        return jnp.dot(h1, w2_ref[...], preferred_element_type=_F32)

    @pl.when(jnp.logical_and(p == 1, t == 0))
    def _():
        s2[...] = jnp.zeros_like(s2)
        q2[...] = jnp.zeros_like(q2)

    @pl.when(p == 1)
    def _():
        y2 = layer2()
        s2[...] += jnp.sum(y2, axis=0, keepdims=True)
        q2[...] += jnp.sum(y2 * y2, axis=0, keepdims=True)

    @pl.when(p == 2)
    def _():
        y2 = layer2()
        s2f = _fold(s2[...], _K, 32)
        q2f = _fold(q2[...], _K, 32)
        m2 = s2f / m_tokens
        v2 = q2f / m_tokens - m2 * m2
        m2p = _tile_lanes(m2, _K)
        sc2p = _tile_lanes(1.0 / jnp.sqrt(v2 + _EPS), _K)
        h2 = jnp.maximum((y2 - m2p) * sc2p * g2_ref[...] + b2_ref[...], 0.0)
        mk = h2[:, 0:32]
        for k in range(1, _K):
            mk = jnp.maximum(mk, h2[:, k * 32:(k + 1) * 32])
        rt = x.shape[0]
        out_ref[pl.ds(t * rt, rt), :] = mk


def _smlp_res(x, wT, g, b, relu):
    y = jnp.dot(x, wT, preferred_element_type=_F32)
    m = jnp.mean(y, axis=0, keepdims=True)
    v = jnp.mean((y - m) * (y - m), axis=0, keepdims=True)
    y = (y - m) / jnp.sqrt(v + _EPS) * g + b
    if relu:
        y = jnp.maximum(y, 0.0)
    return y


def _stage_a1(pts_ref, nbs_ref, na2T, na2g, na2b, na3T, na3g, na3b, f1_ref):
    pts = pts_ref[...]
    lifted = _smlp_res(pts, na2T[...], na2g[...], na2b[...], True)
    x64 = jnp.concatenate([lifted, nbs_ref[...]], axis=1)
    f1_ref[...] = _smlp_res(x64, na3T[...], na3g[...], na3b[...], True)


def _stage_a2(f1_ref, r11T, r11g, r11b, r12T, r12g, r12b,
              r1sT, r1sg, r1sb, f2_ref):
    f1 = f1_ref[...]
    h = _smlp_res(f1, r11T[...], r11g[...], r11b[...], True)
    h2 = _smlp_res(h, r12T[...], r12g[...], r12b[...], False)
    sc = _smlp_res(f1, r1sT[...], r1sg[...], r1sb[...], False)
    f2_ref[...] = jnp.maximum(sc + h2, 0.0)


def _stage_a3(f2_ref, r21T, r21g, r21b, r22T, r22g, r22b,
              f3_ref, gm3_ref, *, b_sz, n_sz):
    f2 = f2_ref[...]
    gm2 = jnp.max(f2.reshape(b_sz, n_sz, 64), axis=1)
    bc2 = jnp.broadcast_to(gm2[:, None, :], (b_sz, n_sz, 64)).reshape(-1, 64)
    x3 = jnp.concatenate([f2, bc2], axis=1)
    h = _smlp_res(x3, r21T[...], r21g[...], r21b[...], True)
    h2 = _smlp_res(h, r22T[...], r22g[...], r22b[...], False)
    f3 = jnp.maximum(x3 + h2, 0.0)
    f3_ref[...] = f3
    gm3_ref[...] = jnp.max(f3.reshape(b_sz, n_sz, 128), axis=1)


def _stage_b(f1_ref, f2_ref, f3_ref, gm3_ref, fuseT, fuseg, fuseb, attT,
             fc1T, fc1g, fc1b, fc2T, fc2g, fc2b, fc3T,
             cdw_ref, sy, qy, zbuf, *, m_tokens, b_sz, n_sz):
    p = pl.program_id(0)
    b = pl.program_id(1)
    gm3b = gm3_ref[pl.ds(b, 1), :]
    x4 = jnp.concatenate(
        [f1_ref[...], f2_ref[...], f3_ref[...],
         jnp.broadcast_to(gm3b, (n_sz, 128))], axis=1)
    y = jnp.dot(x4, fuseT[...], preferred_element_type=_F32)

    @pl.when(jnp.logical_and(p == 0, b == 0))
    def _():
        sy[...] = jnp.zeros_like(sy)
        qy[...] = jnp.zeros_like(qy)

    @pl.when(p == 0)
    def _():
        sy[...] += jnp.sum(y, axis=0, keepdims=True)
        qy[...] += jnp.sum(y * y, axis=0, keepdims=True)

    @pl.when(p == 1)
    def _():
        m = sy[...] / m_tokens
        v = qy[...] / m_tokens - m * m
        f4 = jnp.maximum((y - m) / jnp.sqrt(v + _EPS) * fuseg[...] + fuseb[...],
                         0.0)
        logits = jnp.dot(f4, attT[...], preferred_element_type=_F32)
        mx = jnp.max(logits, axis=0, keepdims=True)
        e = jnp.exp(logits - mx)
        scores = e / jnp.sum(e, axis=0, keepdims=True)
        pooled = jnp.sum(f4 * scores, axis=0, keepdims=True)
        mx4 = jnp.max(f4, axis=0, keepdims=True)
        zbuf[pl.ds(b, 1), :] = jnp.concatenate([mx4, pooled], axis=1)

        @pl.when(b == b_sz - 1)
        def _():
            z = zbuf[...]
            z = _smlp_res(z, fc1T[...], fc1g[...], fc1b[...], True)
            z = _smlp_res(z, fc2T[...], fc2g[...], fc2b[...], True)
            cdw_ref[...] = jnp.dot(z, fc3T[...], preferred_element_type=_F32)


def _row(v):
    return v.reshape(1, -1)


def kernel(pts, params):
    p = params
    b_sz, n_sz, _ = pts.shape
    bn = b_sz * n_sz
    tq = 256
    from functools import partial

    ptsT = jnp.swapaxes(pts, 1, 2)
    feats = pl.pallas_call(
        partial(_knn_kernel, tq=tq),
        grid=(b_sz, n_sz // tq),
        in_specs=[
            pl.BlockSpec((1, tq, 3), lambda b, t: (b, t, 0)),
            pl.BlockSpec((1, n_sz, 3), lambda b, t: (b, 0, 0)),
            pl.BlockSpec((1, 3, n_sz), lambda b, t: (b, 0, 0)),
        ],
        out_specs=pl.BlockSpec((1, tq, 7 * _K), lambda b, t: (b, t, 0)),
        out_shape=jax.ShapeDtypeStruct((b_sz, n_sz, 7 * _K), _F32),
    )(pts, pts, ptsT)

    feats = feats.reshape(bn, 7 * _K)
    eye = jnp.eye(_K, dtype=_F32)
    w1big = jnp.kron(eye, p['na1a_W'].T)
    w2big = jnp.kron(eye, p['na1b_W'].T)
    g1p = _row(jnp.tile(p['na1a_g'], _K))
    b1p = _row(jnp.tile(p['na1a_b'], _K))
    g2p = _row(jnp.tile(p['na1b_g'], _K))
    b2p = _row(jnp.tile(p['na1b_b'], _K))

    rt = 2048
    t_tiles = bn // rt
    nbs = pl.pallas_call(
        partial(_nbrmlp_kernel, m_tokens=float(bn * _K)),
        grid=(3, t_tiles),
        in_specs=[
            pl.BlockSpec((rt, 7 * _K), lambda pp, t: (t, 0)),
            pl.BlockSpec((7 * _K, 16 * _K), lambda pp, t: (0, 0)),
            pl.BlockSpec((1, 16 * _K), lambda pp, t: (0, 0)),
            pl.BlockSpec((1, 16 * _K), lambda pp, t: (0, 0)),
            pl.BlockSpec((16 * _K, 32 * _K), lambda pp, t: (0, 0)),
            pl.BlockSpec((1, 32 * _K), lambda pp, t: (0, 0)),
            pl.BlockSpec((1, 32 * _K), lambda pp, t: (0, 0)),
        ],
        out_specs=pl.BlockSpec((bn, 32), lambda pp, t: (0, 0)),
        out_shape=jax.ShapeDtypeStruct((bn, 32), _F32),
        scratch_shapes=[
            pltpu.VMEM((1, 16 * _K), _F32), pltpu.VMEM((1, 16 * _K), _F32),
            pltpu.VMEM((1, 32 * _K), _F32), pltpu.VMEM((1, 32 * _K), _F32),
        ],
    )(feats, w1big, g1p, b1p, w2big, g2p, b2p)

    full = lambda shp: pl.BlockSpec(shp, lambda: tuple(0 for _ in shp))
    f1 = pl.pallas_call(
        _stage_a1,
        in_specs=[full((bn, 3)), full((bn, 32)),
                  full((3, 32)), full((1, 32)), full((1, 32)),
                  full((64, 32)), full((1, 32)), full((1, 32))],
        out_specs=full((bn, 32)),
        out_shape=jax.ShapeDtypeStruct((bn, 32), _F32),
    )(pts.reshape(bn, 3), nbs,
      p['na2_W'].T, _row(p['na2_g']), _row(p['na2_b']),
      p['na3_W'].T, _row(p['na3_g']), _row(p['na3_b']))

    f2 = pl.pallas_call(
        _stage_a2,
        in_specs=[full((bn, 32)),
                  full((32, 32)), full((1, 32)), full((1, 32)),
                  full((32, 64)), full((1, 64)), full((1, 64)),
                  full((32, 64)), full((1, 64)), full((1, 64))],
        out_specs=full((bn, 64)),
        out_shape=jax.ShapeDtypeStruct((bn, 64), _F32),
    )(f1,
      p['r1_1_W'].T, _row(p['r1_1_g']), _row(p['r1_1_b']),
      p['r1_2_W'].T, _row(p['r1_2_g']), _row(p['r1_2_b']),
      p['r1_s_W'].T, _row(p['r1_s_g']), _row(p['r1_s_b']))

    f3, gm3 = pl.pallas_call(
        partial(_stage_a3, b_sz=b_sz, n_sz=n_sz),
        in_specs=[full((bn, 64)),
                  full((128, 128)), full((1, 128)), full((1, 128)),
                  full((128, 128)), full((1, 128)), full((1, 128))],
        out_specs=[full((bn, 128)), full((b_sz, 128))],
        out_shape=[jax.ShapeDtypeStruct((bn, 128), _F32),
                   jax.ShapeDtypeStruct((b_sz, 128), _F32)],
    )(f2,
      p['r2_1_W'].T, _row(p['r2_1_g']), _row(p['r2_1_b']),
      p['r2_2_W'].T, _row(p['r2_2_g']), _row(p['r2_2_b']))

    wfull = lambda shp: pl.BlockSpec(shp, lambda pp, b: tuple(0 for _ in shp))
    cdw = pl.pallas_call(
        partial(_stage_b, m_tokens=float(bn), b_sz=b_sz, n_sz=n_sz),
        grid=(2, b_sz),
        in_specs=[
            pl.BlockSpec((n_sz, 32), lambda pp, b: (b, 0)),
            pl.BlockSpec((n_sz, 64), lambda pp, b: (b, 0)),
            pl.BlockSpec((n_sz, 128), lambda pp, b: (b, 0)),
            wfull((b_sz, 128)),
            wfull((352, 512)), wfull((1, 512)), wfull((1, 512)),
            wfull((512, 512)),
            wfull((1024, 512)), wfull((1, 512)), wfull((1, 512)),
            wfull((512, 1024)), wfull((1, 1024)), wfull((1, 1024)),
            wfull((1024, 1024)),
        ],
        out_specs=pl.BlockSpec((b_sz, 1024), lambda pp, b: (0, 0)),
        out_shape=jax.ShapeDtypeStruct((b_sz, 1024), _F32),
        scratch_shapes=[
            pltpu.VMEM((1, 512), _F32), pltpu.VMEM((1, 512), _F32),
            pltpu.VMEM((b_sz, 1024), _F32),
        ],
    )(f1, f2, f3, gm3,
      p['fuse_W'].T, _row(p['fuse_g']), _row(p['fuse_b']),
      p['att_W'].T,
      p['fc1_W'].T, _row(p['fc1_g']), _row(p['fc1_b']),
      p['fc2_W'].T, _row(p['fc2_g']), _row(p['fc2_b']),
      p['fc3_W'].T)
    return cdw
